# baseline (speedup 1.0000x reference)
import functools
import math

import jax
import jax.numpy as jnp
from jax.experimental import pallas as pl
from jax.experimental.pallas import tpu as pltpu

BS = 64
L = 64
C = 64
IN = 64
H = 32
KNN = 8
NEG = -1e30

GB = 2
CH = 4


def _ln(z, g, b):
    m = jnp.mean(z, axis=-1, keepdims=True)
    d = z - m
    v = jnp.mean(d * d, axis=-1, keepdims=True)
    return d * jax.lax.rsqrt(v + 1e-5) * g + b


def _dot(a, b):
    return jnp.dot(a, b, preferred_element_type=jnp.float32)


def _bmm(a, b, contract_a, contract_b):
    return jax.lax.dot_general(
        a, b, (((contract_a,), (contract_b,)), ((0,), (0,))),
        preferred_element_type=jnp.float32)


def _graph_kernel(x_ref, fc_w, fc_b,
                  pos0, wqk0, gww0, gb0, lng0, lnb0,
                  pos1, wqk1, gww1, gb1, lng1, lnb1,
                  fw1, fb1, fw2, fb2, flng, flnb,
                  u_ref):
    B = GB * BS
    xc = jnp.concatenate(
        [x_ref[:, :, i * IN:(i + 1) * IN] for i in range(GB)], axis=0)
    h = _dot(xc.reshape(B * L, IN), fc_w[...]) + fc_b[...]

    scale = jnp.float32(1.0 / math.sqrt(H))
    rr = jax.lax.broadcasted_iota(jnp.int32, (L, L), 0)
    cc = jax.lax.broadcasted_iota(jnp.int32, (L, L), 1)
    eye = (rr == cc)[None]

    for (pos, wqk, gww, gb, lng, lnb) in (
            (pos0, wqk0, gww0, gb0, lng0, lnb0),
            (pos1, wqk1, gww1, gb1, lng1, lnb1)):
        h3 = h.reshape(B, L, H)
        hp = (h3 + pos[...][None]).reshape(B * L, H)
        qk = _dot(hp, wqk[...])
        qk3 = qk.reshape(B, L, 2 * H)
        q3 = qk3[:, :, :H]
        k3 = qk3[:, :, H:]
        s = _bmm(q3, k3, 2, 2) * scale

        e_parts, inv_parts, diag_parts = [], [], []
        for g0 in range(0, B, CH):
            sc = s[g0:g0 + CH]
            t = sc
            for i in range(KNN):
                m = jnp.max(t, axis=-1, keepdims=True)
                if i == 0:
                    rowmax = m
                if i < KNN - 1:
                    t = jnp.where(t >= m, NEG, t)
            ec = jnp.where(sc >= m, jnp.exp(sc - rowmax), 0.0)
            den = jnp.sum(ec, axis=-1, keepdims=True)
            dg = jnp.sum(jnp.where(eye, ec, 0.0), axis=-1, keepdims=True)
            inv = 1.0 / den
            e_parts.append(ec)
            inv_parts.append(inv)
            diag_parts.append(dg * inv)
        e = jnp.concatenate(e_parts, axis=0)
        invd = jnp.concatenate(inv_parts, axis=0)
        diag = jnp.concatenate(diag_parts, axis=0)

        xw = _dot(h, gww[...])
        xw3 = xw.reshape(B, L, 2 * H)
        msg = _bmm(e, xw3[:, :, :H], 2, 1) * invd
        out = msg + diag * (xw3[:, :, H:] - xw3[:, :, :H]) + gb[...][None]
        h3 = h3 + jax.nn.relu(out)
        h = _ln(h3.reshape(B * L, H), lng[...], lnb[...])

    a1 = jax.nn.gelu(_dot(h, fw1[...]) + fb1[...])
    z = h + _dot(a1, fw2[...]) + fb2[...]
    z = _ln(z, flng[...], flnb[...])
    u = jnp.tanh(jnp.mean(z.reshape(B, L, H), axis=1))
    for i in range(GB):
        u_ref[i] = u[i * BS:(i + 1) * BS]


def _decoder_kernel(u_ref, dw1, db1, dw2r, db2, o_ref):
    g = jax.nn.gelu(_dot(u_ref[...], dw1[...]) + db1[...])
    o = jnp.sum(g * dw2r[...], axis=-1, keepdims=True) + db2[...]
    o_ref[...] = jnp.broadcast_to(o, (BS, 128))


def _full(shape):
    return pl.BlockSpec(shape, lambda c: (0,) * len(shape))


def kernel(x, p, y, fc_w, fc_b, pos0, wq0, wk0, gw0, gws0, gb0, lng0, lnb0,
           pos1, wq1, wk1, gw1, gws1, gb1, lng1, lnb1,
           fw1, fb1, fw2, fb2, flng, flnb, dw1, db1, dw2, db2):
    del p, y
    fc_b = fc_b.reshape(1, H)
    wqk0 = jnp.concatenate([wq0, wk0], axis=1)
    wqk1 = jnp.concatenate([wq1, wk1], axis=1)
    gww0 = jnp.concatenate([gw0, gws0], axis=1)
    gww1 = jnp.concatenate([gw1, gws1], axis=1)
    gb0 = gb0.reshape(1, H)
    gb1 = gb1.reshape(1, H)
    lng0 = lng0.reshape(1, H)
    lnb0 = lnb0.reshape(1, H)
    lng1 = lng1.reshape(1, H)
    lnb1 = lnb1.reshape(1, H)
    fb1 = fb1.reshape(1, 4 * H)
    fb2 = fb2.reshape(1, H)
    flng = flng.reshape(1, H)
    flnb = flnb.reshape(1, H)
    db1 = db1.reshape(1, H)
    dw2r = dw2.reshape(1, H)
    db2 = db2.reshape(1, 1)

    u = pl.pallas_call(
        _graph_kernel,
        grid=(C // GB,),
        in_specs=[
            pl.BlockSpec((BS, L, GB * IN), lambda c: (0, 0, c)),
            _full((IN, H)), _full((1, H)),
            _full((L, H)), _full((H, 2 * H)), _full((H, 2 * H)),
            _full((1, H)), _full((1, H)), _full((1, H)),
            _full((L, H)), _full((H, 2 * H)), _full((H, 2 * H)),
            _full((1, H)), _full((1, H)), _full((1, H)),
            _full((H, 4 * H)), _full((1, 4 * H)),
            _full((4 * H, H)), _full((1, H)),
            _full((1, H)), _full((1, H)),
        ],
        out_specs=pl.BlockSpec((GB, BS, H), lambda c: (c, 0, 0)),
        out_shape=jax.ShapeDtypeStruct((C, BS, H), jnp.float32),
        compiler_params=pltpu.CompilerParams(
            dimension_semantics=("parallel",)),
    )(x.reshape(BS, L, C * IN), fc_w, fc_b,
      pos0, wqk0, gww0, gb0, lng0, lnb0,
      pos1, wqk1, gww1, gb1, lng1, lnb1,
      fw1, fb1, fw2, fb2, flng, flnb)

    o = pl.pallas_call(
        _decoder_kernel,
        out_shape=jax.ShapeDtypeStruct((BS, 128), jnp.float32),
    )(jnp.transpose(u, (1, 0, 2)).reshape(BS, C * H), dw1, db1, dw2r, db2)
    return o[:, 0]


# speedup vs baseline: 1.4720x; 1.4720x over previous
import functools
import math

import jax
import jax.numpy as jnp
from jax.experimental import pallas as pl
from jax.experimental.pallas import tpu as pltpu

BS = 64
L = 64
C = 64
IN = 64
H = 32
KNN = 8
NEG = -1e30

GB = 2
CH = 4


def _ln(z, g, b):
    m = jnp.mean(z, axis=-1, keepdims=True)
    d = z - m
    v = jnp.mean(d * d, axis=-1, keepdims=True)
    return d * jax.lax.rsqrt(v + 1e-5) * g + b


def _dot(a, b):
    return jnp.dot(a, b, preferred_element_type=jnp.float32)


def _bmm(a, b, contract_a, contract_b):
    return jax.lax.dot_general(
        a, b, (((contract_a,), (contract_b,)), ((0,), (0,))),
        preferred_element_type=jnp.float32)


def _graph_kernel(x_ref, fc_w, fc_b,
                  pos0, wqk0, gww0, gb0, lng0, lnb0,
                  pos1, wqk1, gww1, gb1, lng1, lnb1,
                  fw1, fb1, fw2, fb2, flng, flnb,
                  u_ref):
    B = GB * BS
    xc = jnp.concatenate(
        [x_ref[:, :, i * IN:(i + 1) * IN] for i in range(GB)], axis=0)
    h = _dot(xc.reshape(B * L, IN), fc_w[...]) + fc_b[...]

    scale = jnp.float32(1.0 / math.sqrt(H))
    rr = jax.lax.broadcasted_iota(jnp.int32, (L, L), 0)
    cc = jax.lax.broadcasted_iota(jnp.int32, (L, L), 1)
    eye = (rr == cc)[None]

    for (pos, wqk, gww, gb, lng, lnb) in (
            (pos0, wqk0, gww0, gb0, lng0, lnb0),
            (pos1, wqk1, gww1, gb1, lng1, lnb1)):
        h3 = h.reshape(B, L, H)
        hp = (h3 + pos[...][None]).reshape(B * L, H)
        qk = _dot(hp, wqk[...])
        qk3 = qk.reshape(B, L, 2 * H)
        q3 = qk3[:, :, :H]
        k3 = qk3[:, :, H:]
        s = _bmm(q3, k3, 2, 2) * scale

        t = s
        for i in range(KNN):
            m = jnp.max(t, axis=-1, keepdims=True)
            if i == 0:
                rowmax = m
            if i < KNN - 1:
                t = jnp.where(t >= m, NEG, t)
        e = jnp.where(s >= m, jnp.exp(s - rowmax), 0.0)
        den = jnp.sum(e, axis=-1, keepdims=True)
        dg = jnp.sum(jnp.where(eye, e, 0.0), axis=-1, keepdims=True)
        invd = 1.0 / den
        diag = dg * invd

        xw = _dot(h, gww[...])
        xw3 = xw.reshape(B, L, 2 * H)
        msg = _bmm(e, xw3[:, :, :H], 2, 1) * invd
        out = msg + diag * (xw3[:, :, H:] - xw3[:, :, :H]) + gb[...][None]
        h3 = h3 + jax.nn.relu(out)
        h = _ln(h3.reshape(B * L, H), lng[...], lnb[...])

    a1 = jax.nn.gelu(_dot(h, fw1[...]) + fb1[...])
    z = h + _dot(a1, fw2[...]) + fb2[...]
    z = _ln(z, flng[...], flnb[...])
    u = jnp.tanh(jnp.mean(z.reshape(B, L, H), axis=1))
    for i in range(GB):
        u_ref[i] = u[i * BS:(i + 1) * BS]


def _decoder_kernel(u_ref, dw1, db1, dw2r, db2, o_ref):
    g = jax.nn.gelu(_dot(u_ref[...], dw1[...]) + db1[...])
    o = jnp.sum(g * dw2r[...], axis=-1, keepdims=True) + db2[...]
    o_ref[...] = jnp.broadcast_to(o, (BS, 128))


def _full(shape):
    return pl.BlockSpec(shape, lambda c: (0,) * len(shape))


def kernel(x, p, y, fc_w, fc_b, pos0, wq0, wk0, gw0, gws0, gb0, lng0, lnb0,
           pos1, wq1, wk1, gw1, gws1, gb1, lng1, lnb1,
           fw1, fb1, fw2, fb2, flng, flnb, dw1, db1, dw2, db2):
    del p, y
    fc_b = fc_b.reshape(1, H)
    wqk0 = jnp.concatenate([wq0, wk0], axis=1)
    wqk1 = jnp.concatenate([wq1, wk1], axis=1)
    gww0 = jnp.concatenate([gw0, gws0], axis=1)
    gww1 = jnp.concatenate([gw1, gws1], axis=1)
    gb0 = gb0.reshape(1, H)
    gb1 = gb1.reshape(1, H)
    lng0 = lng0.reshape(1, H)
    lnb0 = lnb0.reshape(1, H)
    lng1 = lng1.reshape(1, H)
    lnb1 = lnb1.reshape(1, H)
    fb1 = fb1.reshape(1, 4 * H)
    fb2 = fb2.reshape(1, H)
    flng = flng.reshape(1, H)
    flnb = flnb.reshape(1, H)
    db1 = db1.reshape(1, H)
    dw2r = dw2.reshape(1, H)
    db2 = db2.reshape(1, 1)

    u = pl.pallas_call(
        _graph_kernel,
        grid=(C // GB,),
        in_specs=[
            pl.BlockSpec((BS, L, GB * IN), lambda c: (0, 0, c)),
            _full((IN, H)), _full((1, H)),
            _full((L, H)), _full((H, 2 * H)), _full((H, 2 * H)),
            _full((1, H)), _full((1, H)), _full((1, H)),
            _full((L, H)), _full((H, 2 * H)), _full((H, 2 * H)),
            _full((1, H)), _full((1, H)), _full((1, H)),
            _full((H, 4 * H)), _full((1, 4 * H)),
            _full((4 * H, H)), _full((1, H)),
            _full((1, H)), _full((1, H)),
        ],
        out_specs=pl.BlockSpec((GB, BS, H), lambda c: (c, 0, 0)),
        out_shape=jax.ShapeDtypeStruct((C, BS, H), jnp.float32),
        compiler_params=pltpu.CompilerParams(
            dimension_semantics=("parallel",)),
    )(x.reshape(BS, L, C * IN), fc_w, fc_b,
      pos0, wqk0, gww0, gb0, lng0, lnb0,
      pos1, wqk1, gww1, gb1, lng1, lnb1,
      fw1, fb1, fw2, fb2, flng, flnb)

    o = pl.pallas_call(
        _decoder_kernel,
        out_shape=jax.ShapeDtypeStruct((BS, 128), jnp.float32),
    )(jnp.transpose(u, (1, 0, 2)).reshape(BS, C * H), dw1, db1, dw2r, db2)
    return o[:, 0]


# speedup vs baseline: 1.4968x; 1.0168x over previous
import functools
import math

import jax
import jax.numpy as jnp
from jax.experimental import pallas as pl
from jax.experimental.pallas import tpu as pltpu

BS = 64
L = 64
C = 64
IN = 64
H = 32
KNN = 8
NEG = -1e30

GB = 2
CH = 4


def _ln(z, g, b):
    m = jnp.mean(z, axis=-1, keepdims=True)
    v = jnp.mean(z * z, axis=-1, keepdims=True) - m * m
    r = jax.lax.rsqrt(v + 1e-5)
    return z * (r * g) + (b - m * r * g)


def _dot(a, b):
    return jnp.dot(a, b, preferred_element_type=jnp.float32)


def _bmm(a, b, contract_a, contract_b):
    return jax.lax.dot_general(
        a, b, (((contract_a,), (contract_b,)), ((0,), (0,))),
        preferred_element_type=jnp.float32)


def _graph_kernel(x_ref, fc_w, fc_b,
                  pos0, wqk0, gww0, gb0, lng0, lnb0,
                  pos1, wqk1, gww1, gb1, lng1, lnb1,
                  fw1, fb1, fw2, fb2, flng, flnb,
                  u_ref):
    B = GB * BS
    xc = jnp.concatenate(
        [x_ref[:, :, i * IN:(i + 1) * IN] for i in range(GB)], axis=0)
    h = _dot(xc.reshape(B * L, IN), fc_w[...]) + fc_b[...]

    scale = jnp.float32(1.0 / math.sqrt(H))
    rr = jax.lax.broadcasted_iota(jnp.int32, (L, L), 0)
    cc = jax.lax.broadcasted_iota(jnp.int32, (L, L), 1)
    eye = (rr == cc)[None]

    for (pos, wqk, gww, gb, lng, lnb) in (
            (pos0, wqk0, gww0, gb0, lng0, lnb0),
            (pos1, wqk1, gww1, gb1, lng1, lnb1)):
        h3 = h.reshape(B, L, H)
        hp = (h3 + pos[...][None]).reshape(B * L, H)
        qk = _dot(hp, wqk[...])
        qk3 = qk.reshape(B, L, 2 * H)
        q3 = qk3[:, :, :H]
        k3 = qk3[:, :, H:]
        s = _bmm(q3, k3, 2, 2)

        t = s
        for i in range(KNN):
            m = jnp.max(t, axis=-1, keepdims=True)
            if i == 0:
                rowmax = m
            if i < KNN - 1:
                t = jnp.where(t >= m, NEG, t)
        e = jnp.where(s >= m, jnp.exp((s - rowmax) * scale), 0.0)
        adj = e / jnp.sum(e, axis=-1, keepdims=True)
        a_off = jnp.where(eye, 0.0, adj)
        diag = jnp.sum(jnp.where(eye, adj, 0.0), axis=-1, keepdims=True)

        xw = _dot(h, gww[...])
        xw3 = xw.reshape(B, L, 2 * H)
        msg = _bmm(a_off, xw3[:, :, :H], 2, 1)
        out = msg + diag * xw3[:, :, H:] + gb[...][None]
        h3 = h3 + jax.nn.relu(out)
        h = _ln(h3.reshape(B * L, H), lng[...], lnb[...])

    a1 = jax.nn.gelu(_dot(h, fw1[...]) + fb1[...])
    z = h + _dot(a1, fw2[...]) + fb2[...]
    z = _ln(z, flng[...], flnb[...])
    u = jnp.tanh(jnp.mean(z.reshape(B, L, H), axis=1))
    for i in range(GB):
        u_ref[i] = u[i * BS:(i + 1) * BS]


def _decoder_kernel(u_ref, dw1, db1, dw2r, db2, o_ref):
    g = jax.nn.gelu(_dot(u_ref[...], dw1[...]) + db1[...])
    o = jnp.sum(g * dw2r[...], axis=-1, keepdims=True) + db2[...]
    o_ref[...] = jnp.broadcast_to(o, (BS, 128))


def _full(shape):
    return pl.BlockSpec(shape, lambda c: (0,) * len(shape))


def kernel(x, p, y, fc_w, fc_b, pos0, wq0, wk0, gw0, gws0, gb0, lng0, lnb0,
           pos1, wq1, wk1, gw1, gws1, gb1, lng1, lnb1,
           fw1, fb1, fw2, fb2, flng, flnb, dw1, db1, dw2, db2):
    del p, y
    fc_b = fc_b.reshape(1, H)
    wqk0 = jnp.concatenate([wq0, wk0], axis=1)
    wqk1 = jnp.concatenate([wq1, wk1], axis=1)
    gww0 = jnp.concatenate([gw0, gws0], axis=1)
    gww1 = jnp.concatenate([gw1, gws1], axis=1)
    gb0 = gb0.reshape(1, H)
    gb1 = gb1.reshape(1, H)
    lng0 = lng0.reshape(1, H)
    lnb0 = lnb0.reshape(1, H)
    lng1 = lng1.reshape(1, H)
    lnb1 = lnb1.reshape(1, H)
    fb1 = fb1.reshape(1, 4 * H)
    fb2 = fb2.reshape(1, H)
    flng = flng.reshape(1, H)
    flnb = flnb.reshape(1, H)
    db1 = db1.reshape(1, H)
    dw2r = dw2.reshape(1, H)
    db2 = db2.reshape(1, 1)

    u = pl.pallas_call(
        _graph_kernel,
        grid=(C // GB,),
        in_specs=[
            pl.BlockSpec((BS, L, GB * IN), lambda c: (0, 0, c)),
            _full((IN, H)), _full((1, H)),
            _full((L, H)), _full((H, 2 * H)), _full((H, 2 * H)),
            _full((1, H)), _full((1, H)), _full((1, H)),
            _full((L, H)), _full((H, 2 * H)), _full((H, 2 * H)),
            _full((1, H)), _full((1, H)), _full((1, H)),
            _full((H, 4 * H)), _full((1, 4 * H)),
            _full((4 * H, H)), _full((1, H)),
            _full((1, H)), _full((1, H)),
        ],
        out_specs=pl.BlockSpec((GB, BS, H), lambda c: (c, 0, 0)),
        out_shape=jax.ShapeDtypeStruct((C, BS, H), jnp.float32),
        compiler_params=pltpu.CompilerParams(
            dimension_semantics=("parallel",)),
    )(x.reshape(BS, L, C * IN), fc_w, fc_b,
      pos0, wqk0, gww0, gb0, lng0, lnb0,
      pos1, wqk1, gww1, gb1, lng1, lnb1,
      fw1, fb1, fw2, fb2, flng, flnb)

    o = pl.pallas_call(
        _decoder_kernel,
        out_shape=jax.ShapeDtypeStruct((BS, 128), jnp.float32),
    )(jnp.transpose(u, (1, 0, 2)).reshape(BS, C * H), dw1, db1, dw2r, db2)
    return o[:, 0]


# speedup vs baseline: 1.5619x; 1.0435x over previous
import functools
import math

import jax
import jax.numpy as jnp
from jax.experimental import pallas as pl
from jax.experimental.pallas import tpu as pltpu

BS = 64
L = 64
C = 64
IN = 64
H = 32
KNN = 8
NEG = -1e30

GB = 2
CH = 4


def _ln(z, g, b):
    m = jnp.mean(z, axis=-1, keepdims=True)
    d = z - m
    v = jnp.mean(d * d, axis=-1, keepdims=True)
    return d * jax.lax.rsqrt(v + 1e-5) * g + b


def _dot(a, b):
    return jnp.dot(a, b, preferred_element_type=jnp.float32)


def _bmm(a, b, contract_a, contract_b):
    return jax.lax.dot_general(
        a, b, (((contract_a,), (contract_b,)), ((0,), (0,))),
        preferred_element_type=jnp.float32)


def _graph_kernel(x_ref, fc_w, fc_b,
                  pos0, wqk0, gww0, gb0, lng0, lnb0,
                  pos1, wqk1, gww1, gb1, lng1, lnb1,
                  fw1, fb1, fw2, fb2, flng, flnb,
                  u_ref):
    B = GB * BS
    xc = jnp.concatenate(
        [x_ref[:, :, i * IN:(i + 1) * IN] for i in range(GB)], axis=0)
    h = _dot(xc.reshape(B * L, IN), fc_w[...]) + fc_b[...]

    scale = jnp.float32(1.0 / math.sqrt(H))
    rr = jax.lax.broadcasted_iota(jnp.int32, (L, L), 0)
    cc = jax.lax.broadcasted_iota(jnp.int32, (L, L), 1)
    eye = (rr == cc)[None]

    for (pos, wqk, gww, gb, lng, lnb) in (
            (pos0, wqk0, gww0, gb0, lng0, lnb0),
            (pos1, wqk1, gww1, gb1, lng1, lnb1)):
        h3 = h.reshape(B, L, H)
        hp = (h3 + pos[...][None]).reshape(B * L, H)
        qk = _dot(hp, wqk[...])
        qk3 = qk.reshape(B, L, 2 * H)
        q3 = qk3[:, :, :H]
        k3 = qk3[:, :, H:]
        s = _bmm(q3, k3, 2, 2) * scale

        t = s
        for i in range(KNN):
            m = jnp.max(t, axis=-1, keepdims=True)
            if i == 0:
                rowmax = m
            if i < KNN - 1:
                t = jnp.where(t >= m, NEG, t)
        e = jnp.where(s >= m, jnp.exp(s - rowmax), 0.0)
        adj = e / jnp.sum(e, axis=-1, keepdims=True)
        a_off = jnp.where(eye, 0.0, adj)
        diag = jnp.sum(jnp.where(eye, adj, 0.0), axis=-1, keepdims=True)

        xw = _dot(h, gww[...])
        xw3 = xw.reshape(B, L, 2 * H)
        msg = _bmm(a_off, xw3[:, :, :H], 2, 1)
        out = msg + diag * xw3[:, :, H:] + gb[...][None]
        h3 = h3 + jax.nn.relu(out)
        h = _ln(h3.reshape(B * L, H), lng[...], lnb[...])

    a1 = jax.nn.gelu(_dot(h, fw1[...]) + fb1[...])
    z = h + _dot(a1, fw2[...]) + fb2[...]
    z = _ln(z, flng[...], flnb[...])
    u = jnp.tanh(jnp.mean(z.reshape(B, L, H), axis=1))
    for i in range(GB):
        u_ref[i] = u[i * BS:(i + 1) * BS]


def _decoder_kernel(u_ref, dw1, db1, dw2r, db2, o_ref):
    g = jax.nn.gelu(_dot(u_ref[...], dw1[...]) + db1[...])
    o = jnp.sum(g * dw2r[...], axis=-1, keepdims=True) + db2[...]
    o_ref[...] = jnp.broadcast_to(o, (BS, 128))


def _full(shape):
    return pl.BlockSpec(shape, lambda c: (0,) * len(shape))


def kernel(x, p, y, fc_w, fc_b, pos0, wq0, wk0, gw0, gws0, gb0, lng0, lnb0,
           pos1, wq1, wk1, gw1, gws1, gb1, lng1, lnb1,
           fw1, fb1, fw2, fb2, flng, flnb, dw1, db1, dw2, db2):
    del p, y
    fc_b = fc_b.reshape(1, H)
    wqk0 = jnp.concatenate([wq0, wk0], axis=1)
    wqk1 = jnp.concatenate([wq1, wk1], axis=1)
    gww0 = jnp.concatenate([gw0, gws0], axis=1)
    gww1 = jnp.concatenate([gw1, gws1], axis=1)
    gb0 = gb0.reshape(1, H)
    gb1 = gb1.reshape(1, H)
    lng0 = lng0.reshape(1, H)
    lnb0 = lnb0.reshape(1, H)
    lng1 = lng1.reshape(1, H)
    lnb1 = lnb1.reshape(1, H)
    fb1 = fb1.reshape(1, 4 * H)
    fb2 = fb2.reshape(1, H)
    flng = flng.reshape(1, H)
    flnb = flnb.reshape(1, H)
    db1 = db1.reshape(1, H)
    dw2r = dw2.reshape(1, H)
    db2 = db2.reshape(1, 1)

    u = pl.pallas_call(
        _graph_kernel,
        grid=(C // GB,),
        in_specs=[
            pl.BlockSpec((BS, L, GB * IN), lambda c: (0, 0, c)),
            _full((IN, H)), _full((1, H)),
            _full((L, H)), _full((H, 2 * H)), _full((H, 2 * H)),
            _full((1, H)), _full((1, H)), _full((1, H)),
            _full((L, H)), _full((H, 2 * H)), _full((H, 2 * H)),
            _full((1, H)), _full((1, H)), _full((1, H)),
            _full((H, 4 * H)), _full((1, 4 * H)),
            _full((4 * H, H)), _full((1, H)),
            _full((1, H)), _full((1, H)),
        ],
        out_specs=pl.BlockSpec((GB, BS, H), lambda c: (c, 0, 0)),
        out_shape=jax.ShapeDtypeStruct((C, BS, H), jnp.float32),
        compiler_params=pltpu.CompilerParams(
            dimension_semantics=("parallel",)),
    )(x.reshape(BS, L, C * IN), fc_w, fc_b,
      pos0, wqk0, gww0, gb0, lng0, lnb0,
      pos1, wqk1, gww1, gb1, lng1, lnb1,
      fw1, fb1, fw2, fb2, flng, flnb)

    o = pl.pallas_call(
        _decoder_kernel,
        out_shape=jax.ShapeDtypeStruct((BS, 128), jnp.float32),
    )(jnp.transpose(u, (1, 0, 2)).reshape(BS, C * H), dw1, db1, dw2r, db2)
    return o[:, 0]


# speedup vs baseline: 1.5640x; 1.0013x over previous
import functools
import math

import jax
import jax.numpy as jnp
from jax.experimental import pallas as pl
from jax.experimental.pallas import tpu as pltpu

BS = 64
L = 64
C = 64
IN = 64
H = 32
KNN = 8
NEG = -1e30

GB = 2
CH = 4


def _ln(z, g, b):
    m = jnp.mean(z, axis=-1, keepdims=True)
    d = z - m
    v = jnp.mean(d * d, axis=-1, keepdims=True)
    return d * jax.lax.rsqrt(v + 1e-5) * g + b


def _dot(a, b):
    return jnp.dot(a, b, preferred_element_type=jnp.float32)


def _bmm(a, b, contract_a, contract_b):
    return jax.lax.dot_general(
        a, b, (((contract_a,), (contract_b,)), ((0,), (0,))),
        preferred_element_type=jnp.float32)


def _graph_kernel(x_ref, fc_w, fc_b,
                  pos0, wqk0, gww0, gb0, lng0, lnb0,
                  pos1, wqk1, gww1, gb1, lng1, lnb1,
                  fw1, fb1, fw2, fb2, flng, flnb,
                  u_ref):
    B = GB * BS
    xc = jnp.concatenate(
        [x_ref[:, :, i * IN:(i + 1) * IN] for i in range(GB)], axis=0)
    h = _dot(xc.reshape(B * L, IN), fc_w[...]) + fc_b[...]

    scale = jnp.float32(1.0 / math.sqrt(H))
    rr = jax.lax.broadcasted_iota(jnp.int32, (L, L), 0)
    cc = jax.lax.broadcasted_iota(jnp.int32, (L, L), 1)
    eye = (rr == cc)[None]

    for (pos, wqk, gww, gb, lng, lnb) in (
            (pos0, wqk0, gww0, gb0, lng0, lnb0),
            (pos1, wqk1, gww1, gb1, lng1, lnb1)):
        h3 = h.reshape(B, L, H)
        hp = (h3 + pos[...][None]).reshape(B * L, H)
        qk = _dot(hp, wqk[...])
        qk3 = qk.reshape(B, L, 2 * H)
        q3 = qk3[:, :, :H]
        k3 = qk3[:, :, H:]
        s = _bmm(q3, k3, 2, 2) * scale

        m = jnp.max(s, axis=-1, keepdims=True)
        rowmax = m
        for i in range(KNN - 1):
            m = jnp.max(jnp.where(s >= m, NEG, s), axis=-1, keepdims=True)
        e = jnp.where(s >= m, jnp.exp(s - rowmax), 0.0)
        adj = e / jnp.sum(e, axis=-1, keepdims=True)
        a_off = jnp.where(eye, 0.0, adj)
        diag = jnp.sum(jnp.where(eye, adj, 0.0), axis=-1, keepdims=True)

        xw = _dot(h, gww[...])
        xw3 = xw.reshape(B, L, 2 * H)
        msg = _bmm(a_off, xw3[:, :, :H], 2, 1)
        out = msg + diag * xw3[:, :, H:] + gb[...][None]
        h3 = h3 + jax.nn.relu(out)
        h = _ln(h3.reshape(B * L, H), lng[...], lnb[...])

    a1 = jax.nn.gelu(_dot(h, fw1[...]) + fb1[...])
    z = h + _dot(a1, fw2[...]) + fb2[...]
    z = _ln(z, flng[...], flnb[...])
    u = jnp.tanh(jnp.mean(z.reshape(B, L, H), axis=1))
    for i in range(GB):
        u_ref[i] = u[i * BS:(i + 1) * BS]


def _decoder_kernel(u_ref, dw1, db1, dw2r, db2, o_ref):
    g = jax.nn.gelu(_dot(u_ref[...], dw1[...]) + db1[...])
    o = jnp.sum(g * dw2r[...], axis=-1, keepdims=True) + db2[...]
    o_ref[...] = jnp.broadcast_to(o, (BS, 128))


def _full(shape):
    return pl.BlockSpec(shape, lambda c: (0,) * len(shape))


def kernel(x, p, y, fc_w, fc_b, pos0, wq0, wk0, gw0, gws0, gb0, lng0, lnb0,
           pos1, wq1, wk1, gw1, gws1, gb1, lng1, lnb1,
           fw1, fb1, fw2, fb2, flng, flnb, dw1, db1, dw2, db2):
    del p, y
    fc_b = fc_b.reshape(1, H)
    wqk0 = jnp.concatenate([wq0, wk0], axis=1)
    wqk1 = jnp.concatenate([wq1, wk1], axis=1)
    gww0 = jnp.concatenate([gw0, gws0], axis=1)
    gww1 = jnp.concatenate([gw1, gws1], axis=1)
    gb0 = gb0.reshape(1, H)
    gb1 = gb1.reshape(1, H)
    lng0 = lng0.reshape(1, H)
    lnb0 = lnb0.reshape(1, H)
    lng1 = lng1.reshape(1, H)
    lnb1 = lnb1.reshape(1, H)
    fb1 = fb1.reshape(1, 4 * H)
    fb2 = fb2.reshape(1, H)
    flng = flng.reshape(1, H)
    flnb = flnb.reshape(1, H)
    db1 = db1.reshape(1, H)
    dw2r = dw2.reshape(1, H)
    db2 = db2.reshape(1, 1)

    u = pl.pallas_call(
        _graph_kernel,
        grid=(C // GB,),
        in_specs=[
            pl.BlockSpec((BS, L, GB * IN), lambda c: (0, 0, c)),
            _full((IN, H)), _full((1, H)),
            _full((L, H)), _full((H, 2 * H)), _full((H, 2 * H)),
            _full((1, H)), _full((1, H)), _full((1, H)),
            _full((L, H)), _full((H, 2 * H)), _full((H, 2 * H)),
            _full((1, H)), _full((1, H)), _full((1, H)),
            _full((H, 4 * H)), _full((1, 4 * H)),
            _full((4 * H, H)), _full((1, H)),
            _full((1, H)), _full((1, H)),
        ],
        out_specs=pl.BlockSpec((GB, BS, H), lambda c: (c, 0, 0)),
        out_shape=jax.ShapeDtypeStruct((C, BS, H), jnp.float32),
        compiler_params=pltpu.CompilerParams(
            dimension_semantics=("parallel",)),
    )(x.reshape(BS, L, C * IN), fc_w, fc_b,
      pos0, wqk0, gww0, gb0, lng0, lnb0,
      pos1, wqk1, gww1, gb1, lng1, lnb1,
      fw1, fb1, fw2, fb2, flng, flnb)

    o = pl.pallas_call(
        _decoder_kernel,
        out_shape=jax.ShapeDtypeStruct((BS, 128), jnp.float32),
    )(jnp.transpose(u, (1, 0, 2)).reshape(BS, C * H), dw1, db1, dw2r, db2)
    return o[:, 0]


# speedup vs baseline: 1.6953x; 1.0840x over previous
import functools
import math

import jax
import jax.numpy as jnp
from jax.experimental import pallas as pl
from jax.experimental.pallas import tpu as pltpu

BS = 64
L = 64
C = 64
IN = 64
H = 32
KNN = 8
NEG = -1e30

GB = 4
CH = 4


def _ln(z, g, b):
    m = jnp.mean(z, axis=-1, keepdims=True)
    d = z - m
    v = jnp.mean(d * d, axis=-1, keepdims=True)
    return d * jax.lax.rsqrt(v + 1e-5) * g + b


def _dot(a, b):
    return jnp.dot(a, b, preferred_element_type=jnp.float32)


def _bmm(a, b, contract_a, contract_b):
    return jax.lax.dot_general(
        a, b, (((contract_a,), (contract_b,)), ((0,), (0,))),
        preferred_element_type=jnp.float32)


def _graph_kernel(x_ref, fc_w, fc_b,
                  pos0, wqk0, gww0, gb0, lng0, lnb0,
                  pos1, wqk1, gww1, gb1, lng1, lnb1,
                  fw1, fb1, fw2, fb2, flng, flnb,
                  u_ref):
    B = GB * BS
    xc = jnp.concatenate(
        [x_ref[:, :, i * IN:(i + 1) * IN] for i in range(GB)], axis=0)
    h = _dot(xc.reshape(B * L, IN), fc_w[...]) + fc_b[...]

    scale = jnp.float32(1.0 / math.sqrt(H))
    rr = jax.lax.broadcasted_iota(jnp.int32, (L, L), 0)
    cc = jax.lax.broadcasted_iota(jnp.int32, (L, L), 1)
    eye = (rr == cc)[None]

    for (pos, wqk, gww, gb, lng, lnb) in (
            (pos0, wqk0, gww0, gb0, lng0, lnb0),
            (pos1, wqk1, gww1, gb1, lng1, lnb1)):
        h3 = h.reshape(B, L, H)
        hp = (h3 + pos[...][None]).reshape(B * L, H)
        qk = _dot(hp, wqk[...])
        qk3 = qk.reshape(B, L, 2 * H)
        q3 = qk3[:, :, :H]
        k3 = qk3[:, :, H:]
        s = _bmm(q3, k3, 2, 2) * scale

        m = jnp.max(s, axis=-1, keepdims=True)
        rowmax = m
        for i in range(KNN - 1):
            m = jnp.max(jnp.where(s >= m, NEG, s), axis=-1, keepdims=True)
        e = jnp.where(s >= m, jnp.exp(s - rowmax), 0.0)
        adj = e / jnp.sum(e, axis=-1, keepdims=True)
        a_off = jnp.where(eye, 0.0, adj)
        diag = jnp.sum(jnp.where(eye, adj, 0.0), axis=-1, keepdims=True)

        xw = _dot(h, gww[...])
        xw3 = xw.reshape(B, L, 2 * H)
        msg = _bmm(a_off, xw3[:, :, :H], 2, 1)
        out = msg + diag * xw3[:, :, H:] + gb[...][None]
        h3 = h3 + jax.nn.relu(out)
        h = _ln(h3.reshape(B * L, H), lng[...], lnb[...])

    a1 = jax.nn.gelu(_dot(h, fw1[...]) + fb1[...])
    z = h + _dot(a1, fw2[...]) + fb2[...]
    z = _ln(z, flng[...], flnb[...])
    u = jnp.tanh(jnp.mean(z.reshape(B, L, H), axis=1))
    for i in range(GB):
        u_ref[i] = u[i * BS:(i + 1) * BS]


def _decoder_kernel(u_ref, dw1, db1, dw2r, db2, o_ref):
    g = jax.nn.gelu(_dot(u_ref[...], dw1[...]) + db1[...])
    o = jnp.sum(g * dw2r[...], axis=-1, keepdims=True) + db2[...]
    o_ref[...] = jnp.broadcast_to(o, (BS, 128))


def _full(shape):
    return pl.BlockSpec(shape, lambda c: (0,) * len(shape))


def kernel(x, p, y, fc_w, fc_b, pos0, wq0, wk0, gw0, gws0, gb0, lng0, lnb0,
           pos1, wq1, wk1, gw1, gws1, gb1, lng1, lnb1,
           fw1, fb1, fw2, fb2, flng, flnb, dw1, db1, dw2, db2):
    del p, y
    fc_b = fc_b.reshape(1, H)
    wqk0 = jnp.concatenate([wq0, wk0], axis=1)
    wqk1 = jnp.concatenate([wq1, wk1], axis=1)
    gww0 = jnp.concatenate([gw0, gws0], axis=1)
    gww1 = jnp.concatenate([gw1, gws1], axis=1)
    gb0 = gb0.reshape(1, H)
    gb1 = gb1.reshape(1, H)
    lng0 = lng0.reshape(1, H)
    lnb0 = lnb0.reshape(1, H)
    lng1 = lng1.reshape(1, H)
    lnb1 = lnb1.reshape(1, H)
    fb1 = fb1.reshape(1, 4 * H)
    fb2 = fb2.reshape(1, H)
    flng = flng.reshape(1, H)
    flnb = flnb.reshape(1, H)
    db1 = db1.reshape(1, H)
    dw2r = dw2.reshape(1, H)
    db2 = db2.reshape(1, 1)

    u = pl.pallas_call(
        _graph_kernel,
        grid=(C // GB,),
        in_specs=[
            pl.BlockSpec((BS, L, GB * IN), lambda c: (0, 0, c)),
            _full((IN, H)), _full((1, H)),
            _full((L, H)), _full((H, 2 * H)), _full((H, 2 * H)),
            _full((1, H)), _full((1, H)), _full((1, H)),
            _full((L, H)), _full((H, 2 * H)), _full((H, 2 * H)),
            _full((1, H)), _full((1, H)), _full((1, H)),
            _full((H, 4 * H)), _full((1, 4 * H)),
            _full((4 * H, H)), _full((1, H)),
            _full((1, H)), _full((1, H)),
        ],
        out_specs=pl.BlockSpec((GB, BS, H), lambda c: (c, 0, 0)),
        out_shape=jax.ShapeDtypeStruct((C, BS, H), jnp.float32),
        compiler_params=pltpu.CompilerParams(
            dimension_semantics=("parallel",),
            vmem_limit_bytes=64 * 1024 * 1024),
    )(x.reshape(BS, L, C * IN), fc_w, fc_b,
      pos0, wqk0, gww0, gb0, lng0, lnb0,
      pos1, wqk1, gww1, gb1, lng1, lnb1,
      fw1, fb1, fw2, fb2, flng, flnb)

    o = pl.pallas_call(
        _decoder_kernel,
        out_shape=jax.ShapeDtypeStruct((BS, 128), jnp.float32),
    )(jnp.transpose(u, (1, 0, 2)).reshape(BS, C * H), dw1, db1, dw2r, db2)
    return o[:, 0]
